# baseline (speedup 1.0000x reference)
import functools

import jax
import jax.numpy as jnp
from jax.experimental import pallas as pl
from jax.experimental.pallas import tpu as pltpu

_EPS = 1e-9
_MIN_EXPERT_CAPACITY = 4


def _routing_kernel(x_ref, w_ref, tri_ref, meta_ref, counts_ref, loss_ref,
                    *, cap, loss_scale):
    E = w_ref.shape[1]
    T = x_ref.shape[1]

    x = x_ref[0]
    logits = jnp.dot(x, w_ref[...], preferred_element_type=jnp.float32)
    m = jnp.max(logits, axis=1, keepdims=True)
    ex = jnp.exp(logits - m)
    probs = ex / jnp.sum(ex, axis=1, keepdims=True)

    iota_e = jax.lax.broadcasted_iota(jnp.int32, (T, E), 1)
    g1 = jnp.max(probs, axis=1, keepdims=True)
    idx1 = jnp.min(jnp.where(probs == g1, iota_e, E), axis=1, keepdims=True)
    mask1 = (iota_e == idx1).astype(jnp.float32)

    probs2 = probs * (1.0 - mask1)
    g2 = jnp.max(probs2, axis=1, keepdims=True)
    idx2 = jnp.min(jnp.where(probs2 == g2, iota_e, E), axis=1, keepdims=True)
    mask2 = (iota_e == idx2).astype(jnp.float32)

    denom = g1 + g2 + _EPS
    g1n = g1 / denom
    g2n = g2 / denom

    tri = tri_ref[...]
    cum1 = jnp.dot(tri, mask1.astype(jnp.bfloat16),
                   preferred_element_type=jnp.float32)
    cum2 = jnp.dot(tri, mask2.astype(jnp.bfloat16),
                   preferred_element_type=jnp.float32)
    pos1 = jnp.sum(cum1 * mask1, axis=1, keepdims=True)
    cum2t = jnp.sum(cum2 * mask2, axis=1, keepdims=True)
    keep1 = (pos1 < float(cap)).astype(jnp.float32)
    g1f = g1n * keep1

    feat = jnp.concatenate(
        [idx1.astype(jnp.float32), pos1, g1f,
         idx2.astype(jnp.float32), cum2t, g2n,
         jnp.zeros((T, 2), jnp.float32)], axis=1)
    meta_ref[0, 0] = feat

    total1 = jnp.sum(mask1, axis=0, keepdims=True)
    psum = jnp.sum(probs, axis=0, keepdims=True)
    counts_ref[0] = total1
    loss_ref[0, 0, 0] = jnp.sum(total1 * psum) * loss_scale


def _materialize_kernel(meta_ref, counts_ref, comb_ref, *, cap):
    E = counts_ref.shape[2]
    feat = meta_ref[0, 0]
    T = feat.shape[0]
    idx1 = feat[:, 0:1]
    pos1 = feat[:, 1:2]
    g1f = feat[:, 2:3]
    idx2 = feat[:, 3:4]
    cum2t = feat[:, 4:5]
    g2n = feat[:, 5:6]

    m1c = jnp.minimum(counts_ref[0], float(cap))
    iota_e = jax.lax.broadcasted_iota(jnp.int32, (T, E), 1).astype(jnp.float32)
    oh_e2 = (iota_e == idx2).astype(jnp.float32)
    pos2 = cum2t + jnp.sum(oh_e2 * m1c, axis=1, keepdims=True)
    keep2 = (pos2 < float(cap)).astype(jnp.float32)
    g2f = g2n * keep2

    p1 = idx1 * float(cap) + pos1
    p2 = idx2 * float(cap) + pos2
    iota = jax.lax.broadcasted_iota(
        jnp.int32, (T, E * cap), 1).astype(jnp.float32)
    zero = jnp.zeros((), jnp.float32)
    comb = (jnp.where(iota == p1, g1f, zero)
            + jnp.where(iota == p2, g2f, zero))
    comb_ref[0] = comb


@jax.jit
def kernel(x, w_gating):
    B, N, D = x.shape
    E = w_gating.shape[1]
    cap = int((N * 1.25) / E)
    cap = max(min(N, cap), _MIN_EXPERT_CAPACITY)

    rr = jax.lax.broadcasted_iota(jnp.int32, (N, N), 0)
    cc = jax.lax.broadcasted_iota(jnp.int32, (N, N), 1)
    tri = (cc < rr).astype(jnp.bfloat16)

    meta, counts, loss = pl.pallas_call(
        functools.partial(_routing_kernel, cap=cap,
                          loss_scale=float(E) / float(B) / float(N) / float(N)),
        grid=(B,),
        in_specs=[
            pl.BlockSpec((1, N, D), lambda b: (b, 0, 0)),
            pl.BlockSpec((D, E), lambda b: (0, 0)),
            pl.BlockSpec((N, N), lambda b: (0, 0)),
        ],
        out_specs=[
            pl.BlockSpec((1, 1, N, 8), lambda b: (b, 0, 0, 0)),
            pl.BlockSpec((1, 1, E), lambda b: (b, 0, 0)),
            pl.BlockSpec((1, 1, 1), lambda b: (b, 0, 0),
                         memory_space=pltpu.SMEM),
        ],
        out_shape=[
            jax.ShapeDtypeStruct((B, 1, N, 8), jnp.float32),
            jax.ShapeDtypeStruct((B, 1, E), jnp.float32),
            jax.ShapeDtypeStruct((B, 1, 1), jnp.float32),
        ],
        compiler_params=pltpu.CompilerParams(
            dimension_semantics=("arbitrary",)),
    )(x, w_gating, tri)

    T = 512
    NB = N // T
    combine = pl.pallas_call(
        functools.partial(_materialize_kernel, cap=cap),
        grid=(B, NB),
        in_specs=[
            pl.BlockSpec((1, 1, T, 8), lambda b, nb: (b, 0, nb, 0)),
            pl.BlockSpec((1, 1, E), lambda b, nb: (b, 0, 0)),
        ],
        out_specs=pl.BlockSpec((1, T, E * cap), lambda b, nb: (b, nb, 0)),
        out_shape=jax.ShapeDtypeStruct((B, N, E * cap), jnp.float32),
        compiler_params=pltpu.CompilerParams(
            dimension_semantics=("parallel", "parallel")),
    )(meta, counts)

    dispatch = (combine != 0.0).astype(jnp.float32).reshape(B, N, E, cap)
    combine = combine.reshape(B, N, E, cap)
    return dispatch, combine, jnp.sum(loss)


# speedup vs baseline: 1.0446x; 1.0446x over previous
import functools

import jax
import jax.numpy as jnp
from jax.experimental import pallas as pl
from jax.experimental.pallas import tpu as pltpu

_EPS = 1e-9
_MIN_EXPERT_CAPACITY = 4


def _routing_kernel(x_ref, w_ref, tri_ref, meta_ref, counts_ref, loss_ref,
                    *, cap, loss_scale):
    E = w_ref.shape[1]
    T = x_ref.shape[1]

    x = x_ref[0]
    logits = jnp.dot(x, w_ref[...], preferred_element_type=jnp.float32)
    m = jnp.max(logits, axis=1, keepdims=True)
    ex = jnp.exp(logits - m)
    probs = ex / jnp.sum(ex, axis=1, keepdims=True)

    iota_e = jax.lax.broadcasted_iota(jnp.int32, (T, E), 1)
    g1 = jnp.max(probs, axis=1, keepdims=True)
    idx1 = jnp.min(jnp.where(probs == g1, iota_e, E), axis=1, keepdims=True)
    mask1 = (iota_e == idx1).astype(jnp.float32)

    probs2 = probs * (1.0 - mask1)
    g2 = jnp.max(probs2, axis=1, keepdims=True)
    idx2 = jnp.min(jnp.where(probs2 == g2, iota_e, E), axis=1, keepdims=True)
    mask2 = (iota_e == idx2).astype(jnp.float32)

    denom = g1 + g2 + _EPS
    g1n = g1 / denom
    g2n = g2 / denom

    tri = tri_ref[...]
    cum1 = jnp.dot(tri, mask1.astype(jnp.bfloat16),
                   preferred_element_type=jnp.float32)
    cum2 = jnp.dot(tri, mask2.astype(jnp.bfloat16),
                   preferred_element_type=jnp.float32)
    pos1 = jnp.sum(cum1 * mask1, axis=1, keepdims=True)
    cum2t = jnp.sum(cum2 * mask2, axis=1, keepdims=True)
    keep1 = (pos1 < float(cap)).astype(jnp.float32)
    g1f = g1n * keep1

    feat = jnp.concatenate(
        [idx1.astype(jnp.float32), pos1, g1f,
         idx2.astype(jnp.float32), cum2t, g2n,
         jnp.zeros((T, 2), jnp.float32)], axis=1)
    meta_ref[0, 0] = feat

    total1 = jnp.sum(mask1, axis=0, keepdims=True)
    psum = jnp.sum(probs, axis=0, keepdims=True)
    counts_ref[0] = total1
    loss_ref[0, 0, 0] = jnp.sum(total1 * psum) * loss_scale


def _materialize_kernel(meta_ref, counts_ref, comb_ref, disp_ref, *, cap):
    E = counts_ref.shape[2]
    feat = meta_ref[0, 0]
    T = feat.shape[0]
    idx1 = feat[:, 0:1]
    pos1 = feat[:, 1:2]
    g1f = feat[:, 2:3]
    idx2 = feat[:, 3:4]
    cum2t = feat[:, 4:5]
    g2n = feat[:, 5:6]

    m1c = jnp.minimum(counts_ref[0], float(cap))
    iota_e = jax.lax.broadcasted_iota(jnp.int32, (T, E), 1).astype(jnp.float32)
    oh_e2 = (iota_e == idx2).astype(jnp.float32)
    pos2 = cum2t + jnp.sum(oh_e2 * m1c, axis=1, keepdims=True)
    keep2 = (pos2 < float(cap)).astype(jnp.float32)
    g2f = g2n * keep2

    p1 = idx1 * float(cap) + pos1
    p2 = idx2 * float(cap) + pos2
    iota = jax.lax.broadcasted_iota(
        jnp.int32, (T, E * cap), 1).astype(jnp.float32)
    zero = jnp.zeros((), jnp.float32)
    comb = (jnp.where(iota == p1, g1f, zero)
            + jnp.where(iota == p2, g2f, zero))
    comb_ref[0] = comb
    disp_ref[0] = (comb != 0.0).astype(jnp.float32)


@jax.jit
def kernel(x, w_gating):
    B, N, D = x.shape
    E = w_gating.shape[1]
    cap = int((N * 1.25) / E)
    cap = max(min(N, cap), _MIN_EXPERT_CAPACITY)

    rr = jax.lax.broadcasted_iota(jnp.int32, (N, N), 0)
    cc = jax.lax.broadcasted_iota(jnp.int32, (N, N), 1)
    tri = (cc < rr).astype(jnp.bfloat16)

    meta, counts, loss = pl.pallas_call(
        functools.partial(_routing_kernel, cap=cap,
                          loss_scale=float(E) / float(B) / float(N) / float(N)),
        grid=(B,),
        in_specs=[
            pl.BlockSpec((1, N, D), lambda b: (b, 0, 0)),
            pl.BlockSpec((D, E), lambda b: (0, 0)),
            pl.BlockSpec((N, N), lambda b: (0, 0)),
        ],
        out_specs=[
            pl.BlockSpec((1, 1, N, 8), lambda b: (b, 0, 0, 0)),
            pl.BlockSpec((1, 1, E), lambda b: (b, 0, 0)),
            pl.BlockSpec((1, 1, 1), lambda b: (b, 0, 0),
                         memory_space=pltpu.SMEM),
        ],
        out_shape=[
            jax.ShapeDtypeStruct((B, 1, N, 8), jnp.float32),
            jax.ShapeDtypeStruct((B, 1, E), jnp.float32),
            jax.ShapeDtypeStruct((B, 1, 1), jnp.float32),
        ],
        compiler_params=pltpu.CompilerParams(
            dimension_semantics=("arbitrary",)),
    )(x, w_gating, tri)

    T = 1024
    NB = N // T
    combine, dispatch = pl.pallas_call(
        functools.partial(_materialize_kernel, cap=cap),
        grid=(B, NB),
        in_specs=[
            pl.BlockSpec((1, 1, T, 8), lambda b, nb: (b, 0, nb, 0)),
            pl.BlockSpec((1, 1, E), lambda b, nb: (b, 0, 0)),
        ],
        out_specs=[
            pl.BlockSpec((1, T, E * cap), lambda b, nb: (b, nb, 0)),
            pl.BlockSpec((1, T, E * cap), lambda b, nb: (b, nb, 0)),
        ],
        out_shape=[
            jax.ShapeDtypeStruct((B, N, E * cap), jnp.float32),
            jax.ShapeDtypeStruct((B, N, E * cap), jnp.float32),
        ],
        compiler_params=pltpu.CompilerParams(
            dimension_semantics=("parallel", "parallel")),
    )(meta, counts)

    combine = combine.reshape(B, N, E, cap)
    dispatch = dispatch.reshape(B, N, E, cap)
    return dispatch, combine, jnp.sum(loss)


# speedup vs baseline: 1.0541x; 1.0090x over previous
import functools

import jax
import jax.numpy as jnp
from jax.experimental import pallas as pl
from jax.experimental.pallas import tpu as pltpu

_EPS = 1e-9
_MIN_EXPERT_CAPACITY = 4


def _routing_kernel(x_ref, w_ref, tri_ref, meta_ref, counts_ref, loss_ref,
                    *, cap, loss_scale):
    E = w_ref.shape[1]
    T = x_ref.shape[1]

    x = x_ref[0]
    logits = jnp.dot(x, w_ref[...], preferred_element_type=jnp.float32)
    m = jnp.max(logits, axis=1, keepdims=True)
    ex = jnp.exp(logits - m)
    probs = ex / jnp.sum(ex, axis=1, keepdims=True)

    iota_e = jax.lax.broadcasted_iota(jnp.int32, (T, E), 1)
    g1 = jnp.max(probs, axis=1, keepdims=True)
    idx1 = jnp.min(jnp.where(probs == g1, iota_e, E), axis=1, keepdims=True)
    mask1 = (iota_e == idx1).astype(jnp.float32)

    probs2 = probs * (1.0 - mask1)
    g2 = jnp.max(probs2, axis=1, keepdims=True)
    idx2 = jnp.min(jnp.where(probs2 == g2, iota_e, E), axis=1, keepdims=True)
    mask2 = (iota_e == idx2).astype(jnp.float32)

    denom = g1 + g2 + _EPS
    g1n = g1 / denom
    g2n = g2 / denom

    tri = tri_ref[...]
    cum1 = jnp.dot(tri, mask1.astype(jnp.bfloat16),
                   preferred_element_type=jnp.float32)
    cum2 = jnp.dot(tri, mask2.astype(jnp.bfloat16),
                   preferred_element_type=jnp.float32)
    pos1 = jnp.sum(cum1 * mask1, axis=1, keepdims=True)
    cum2t = jnp.sum(cum2 * mask2, axis=1, keepdims=True)
    keep1 = (pos1 < float(cap)).astype(jnp.float32)
    g1f = g1n * keep1

    feat = jnp.concatenate(
        [idx1.astype(jnp.float32), pos1, g1f,
         idx2.astype(jnp.float32), cum2t, g2n,
         jnp.zeros((T, 2), jnp.float32)], axis=1)
    meta_ref[0, 0] = feat

    total1 = jnp.sum(mask1, axis=0, keepdims=True)
    psum = jnp.sum(probs, axis=0, keepdims=True)
    counts_ref[0] = total1
    loss_ref[0, 0, 0] = jnp.sum(total1 * psum) * loss_scale


def _materialize_kernel(meta_ref, counts_ref, comb_ref, disp_ref, *, cap):
    E = counts_ref.shape[2]
    feat = meta_ref[0, 0]
    T = feat.shape[0]
    idx1 = feat[:, 0:1]
    pos1 = feat[:, 1:2]
    g1f = feat[:, 2:3]
    idx2 = feat[:, 3:4]
    cum2t = feat[:, 4:5]
    g2n = feat[:, 5:6]

    m1c = jnp.minimum(counts_ref[0], float(cap))
    iota_e = jax.lax.broadcasted_iota(jnp.int32, (T, E), 1).astype(jnp.float32)
    oh_e2 = (iota_e == idx2).astype(jnp.float32)
    pos2 = cum2t + jnp.sum(oh_e2 * m1c, axis=1, keepdims=True)
    keep2 = (pos2 < float(cap)).astype(jnp.float32)
    g2f = g2n * keep2

    p1 = idx1 * float(cap) + pos1
    p2 = idx2 * float(cap) + pos2
    iota = jax.lax.broadcasted_iota(
        jnp.int32, (T, E * cap), 1).astype(jnp.float32)
    zero = jnp.zeros((), jnp.float32)
    comb = (jnp.where(iota == p1, g1f, zero)
            + jnp.where(iota == p2, g2f, zero))
    comb_ref[0] = comb
    disp_ref[0] = (comb != 0.0).astype(jnp.float32)


@jax.jit
def kernel(x, w_gating):
    B, N, D = x.shape
    E = w_gating.shape[1]
    cap = int((N * 1.25) / E)
    cap = max(min(N, cap), _MIN_EXPERT_CAPACITY)

    rr = jax.lax.broadcasted_iota(jnp.int32, (N, N), 0)
    cc = jax.lax.broadcasted_iota(jnp.int32, (N, N), 1)
    tri = (cc < rr).astype(jnp.bfloat16)

    meta, counts, loss = pl.pallas_call(
        functools.partial(_routing_kernel, cap=cap,
                          loss_scale=float(E) / float(B) / float(N) / float(N)),
        grid=(B,),
        in_specs=[
            pl.BlockSpec((1, N, D), lambda b: (b, 0, 0)),
            pl.BlockSpec((D, E), lambda b: (0, 0)),
            pl.BlockSpec((N, N), lambda b: (0, 0)),
        ],
        out_specs=[
            pl.BlockSpec((1, 1, N, 8), lambda b: (b, 0, 0, 0)),
            pl.BlockSpec((1, 1, E), lambda b: (b, 0, 0)),
            pl.BlockSpec((1, 1, 1), lambda b: (b, 0, 0),
                         memory_space=pltpu.SMEM),
        ],
        out_shape=[
            jax.ShapeDtypeStruct((B, 1, N, 8), jnp.float32),
            jax.ShapeDtypeStruct((B, 1, E), jnp.float32),
            jax.ShapeDtypeStruct((B, 1, 1), jnp.float32),
        ],
        compiler_params=pltpu.CompilerParams(
            dimension_semantics=("arbitrary",)),
    )(x, w_gating, tri)

    T = 512
    NB = N // T
    combine, dispatch = pl.pallas_call(
        functools.partial(_materialize_kernel, cap=cap),
        grid=(B, NB),
        in_specs=[
            pl.BlockSpec((1, 1, T, 8), lambda b, nb: (b, 0, nb, 0)),
            pl.BlockSpec((1, 1, E), lambda b, nb: (b, 0, 0)),
        ],
        out_specs=[
            pl.BlockSpec((1, T, E * cap), lambda b, nb: (b, nb, 0)),
            pl.BlockSpec((1, T, E * cap), lambda b, nb: (b, nb, 0)),
        ],
        out_shape=[
            jax.ShapeDtypeStruct((B, N, E * cap), jnp.float32),
            jax.ShapeDtypeStruct((B, N, E * cap), jnp.float32),
        ],
        compiler_params=pltpu.CompilerParams(
            dimension_semantics=("parallel", "parallel")),
    )(meta, counts)

    combine = combine.reshape(B, N, E, cap)
    dispatch = dispatch.reshape(B, N, E, cap)
    return dispatch, combine, jnp.sum(loss)
